# Initial kernel scaffold; baseline (speedup 1.0000x reference)
import functools

import jax
import jax.numpy as jnp
from jax import lax
from jax.experimental import pallas as pl
from jax.experimental.pallas import tpu as pltpu
from jax.experimental.pallas import tpu_sc as plsc

VOCAB = 1000000
EMBED = 64
BATCH = 16384
POS = 10
NEG = 50

NC = 2
NS = 16
NW = NC * NS
L = 16

B_PER_W = BATCH // NW
CB = 8
STEPS = B_PER_W // CB
NEG_CHUNKS = CB * NEG // 80


def _sc_dots_kernel(cen_hbm, pos_hbm, neg_hbm, in_hbm, out_hbm, dots_hbm,
                    idx_c, idx_p, idx_n, cen_rows, pos_rows, neg_rows,
                    dots_v, sem):
    wid = lax.axis_index("s") * NC + lax.axis_index("c")
    lane = lax.broadcasted_iota(jnp.int32, (L,), 0)

    def step(s, carry):
        b0 = wid * B_PER_W + s * CB
        pltpu.sync_copy(cen_hbm.at[pl.ds(b0, CB)], idx_c)
        pltpu.sync_copy(pos_hbm.at[pl.ds(b0 * POS, CB * POS)], idx_p)
        for k in range(NEG_CHUNKS):
            pltpu.sync_copy(neg_hbm.at[pl.ds(b0 * NEG + 80 * k, 80)],
                            idx_n.at[k])
        cps = [pltpu.async_copy(in_hbm.at[idx_c], cen_rows, sem),
               pltpu.async_copy(out_hbm.at[idx_p], pos_rows, sem)]
        for k in range(NEG_CHUNKS):
            cps.append(pltpu.async_copy(out_hbm.at[idx_n.at[k]],
                                        neg_rows.at[pl.ds(80 * k, 80)], sem))
        for cp in cps:
            cp.wait()

        def item(b, carry2):
            c = [cen_rows[b, pl.ds(16 * k, L)] for k in range(4)]
            d = [jnp.zeros((L,), jnp.float32) for _ in range(4)]
            for j in range(POS):
                row = b * POS + j
                acc = pos_rows[row, pl.ds(0, L)] * c[0]
                for k in range(1, 4):
                    acc = acc + pos_rows[row, pl.ds(16 * k, L)] * c[k]
                dot = jnp.sum(acc)
                g, ln = divmod(j, L)
                d[g] = jnp.where(lane == ln, dot, d[g])
            for j in range(NEG):
                row = b * NEG + j
                acc = neg_rows[row, pl.ds(0, L)] * c[0]
                for k in range(1, 4):
                    acc = acc + neg_rows[row, pl.ds(16 * k, L)] * c[k]
                dot = jnp.sum(acc)
                g, ln = divmod(POS + j, L)
                d[g] = jnp.where(lane == ln, dot, d[g])
            for g in range(4):
                dots_v[b, pl.ds(16 * g, L)] = d[g]
            return carry2

        lax.fori_loop(0, CB, item, 0)
        pltpu.sync_copy(dots_v, dots_hbm.at[pl.ds(b0, CB)])
        return carry

    lax.fori_loop(0, STEPS, step, 0)


def _tc_loss_kernel(dots_ref, out_ref):
    x = dots_ref[...]
    lane = lax.broadcasted_iota(jnp.int32, x.shape, 1)
    sign = jnp.where(lane < POS, 1.0, -1.0).astype(jnp.float32)
    y = jax.nn.log_sigmoid(x * sign)
    y = jnp.where(lane < POS + NEG, y, 0.0)
    out_ref[...] = -jnp.sum(y, axis=1)


def kernel(cen_tensor, pos_tensors, neg_tensors, in_table, out_table):
    pos_flat = pos_tensors.reshape(-1)
    neg_flat = neg_tensors.reshape(-1)

    mesh = plsc.VectorSubcoreMesh(core_axis_name="c", subcore_axis_name="s")
    sc_call = functools.partial(
        pl.kernel, mesh=mesh,
        out_type=jax.ShapeDtypeStruct((BATCH, EMBED), jnp.float32),
        scratch_types=[
            pltpu.VMEM((CB,), jnp.int32),
            pltpu.VMEM((CB * POS,), jnp.int32),
            pltpu.VMEM((NEG_CHUNKS, 80), jnp.int32),
            pltpu.VMEM((CB, EMBED), jnp.float32),
            pltpu.VMEM((CB * POS, EMBED), jnp.float32),
            pltpu.VMEM((CB * NEG, EMBED), jnp.float32),
            pltpu.VMEM((CB, EMBED), jnp.float32),
            pltpu.SemaphoreType.DMA,
        ],
    )(_sc_dots_kernel)
    dots = sc_call(cen_tensor, pos_flat, neg_flat, in_table, out_table)

    return pl.pallas_call(
        _tc_loss_kernel,
        out_shape=jax.ShapeDtypeStruct((BATCH,), jnp.float32),
    )(dots)


# speedup vs baseline: 3.2875x; 3.2875x over previous
import functools

import jax
import jax.numpy as jnp
from jax import lax
from jax.experimental import pallas as pl
from jax.experimental.pallas import tpu as pltpu
from jax.experimental.pallas import tpu_sc as plsc

VOCAB = 1000000
EMBED = 64
BATCH = 16384
POS = 10
NEG = 50

NC = 2
NS = 16
NW = NC * NS
L = 16

B_PER_W = BATCH // NW
CB = 8
STEPS = B_PER_W // CB
NEG_CHUNKS = CB * NEG // 80


def _sc_dots_kernel(cen_hbm, pos_hbm, neg_hbm, in_hbm, out_hbm, dots_hbm,
                    idx_c, idx_p, idx_n, cen_rows, pos_rows, neg_rows,
                    dots_v, sem):
    wid = lax.axis_index("s") * NC + lax.axis_index("c")
    lane = lax.broadcasted_iota(jnp.int32, (L,), 0)

    def step(s, carry):
        b0 = wid * B_PER_W + s * CB
        pltpu.sync_copy(cen_hbm.at[pl.ds(b0, CB)], idx_c)
        pltpu.sync_copy(pos_hbm.at[pl.ds(b0 * POS, CB * POS)], idx_p)
        for k in range(NEG_CHUNKS):
            pltpu.sync_copy(neg_hbm.at[pl.ds(b0 * NEG + 80 * k, 80)],
                            idx_n.at[k])
        cps = [pltpu.async_copy(in_hbm.at[idx_c], cen_rows, sem),
               pltpu.async_copy(out_hbm.at[idx_p], pos_rows, sem)]
        for k in range(NEG_CHUNKS):
            cps.append(pltpu.async_copy(out_hbm.at[idx_n.at[k]],
                                        neg_rows.at[pl.ds(80 * k, 80)], sem))
        for cp in cps:
            cp.wait()

        def item(b, carry2):
            c = [cen_rows[b, pl.ds(16 * k, L)] for k in range(4)]
            d = [jnp.zeros((L,), jnp.float32) for _ in range(4)]
            for j in range(POS):
                row = b * POS + j
                acc = pos_rows[row, pl.ds(0, L)] * c[0]
                for k in range(1, 4):
                    acc = acc + pos_rows[row, pl.ds(16 * k, L)] * c[k]
                dot = jnp.sum(acc)
                g, ln = divmod(j, L)
                d[g] = jnp.where(lane == ln, dot, d[g])
            for j in range(NEG):
                row = b * NEG + j
                acc = neg_rows[row, pl.ds(0, L)] * c[0]
                for k in range(1, 4):
                    acc = acc + neg_rows[row, pl.ds(16 * k, L)] * c[k]
                dot = jnp.sum(acc)
                g, ln = divmod(POS + j, L)
                d[g] = jnp.where(lane == ln, dot, d[g])
            for g in range(4):
                dots_v[b, pl.ds(16 * g, L)] = d[g]
            return carry2

        lax.fori_loop(0, CB, item, 0)
        pltpu.sync_copy(dots_v, dots_hbm.at[pl.ds(b0, CB)])
        return carry

    lax.fori_loop(0, STEPS, step, 0)


def _tc_loss_kernel(dots_ref, out_ref):
    x = dots_ref[...]
    lane = lax.broadcasted_iota(jnp.int32, x.shape, 1)
    sign = jnp.where(lane < POS, 1.0, -1.0).astype(jnp.float32)
    y = jax.nn.log_sigmoid(x * sign)
    y = jnp.where(lane < POS + NEG, y, 0.0)
    out_ref[...] = -jnp.sum(y, axis=1)


def kernel(cen_tensor, pos_tensors, neg_tensors, in_table, out_table):
    pos_flat = pos_tensors.reshape(-1)
    neg_flat = neg_tensors.reshape(-1)

    mesh = plsc.VectorSubcoreMesh(core_axis_name="c", subcore_axis_name="s")
    sc_call = functools.partial(
        pl.kernel, mesh=mesh,
        compiler_params=pltpu.CompilerParams(needs_layout_passes=False,
                                             use_tc_tiling_on_sc=False),
        out_type=jax.ShapeDtypeStruct((BATCH, EMBED), jnp.float32),
        scratch_types=[
            pltpu.VMEM((CB,), jnp.int32),
            pltpu.VMEM((CB * POS,), jnp.int32),
            pltpu.VMEM((NEG_CHUNKS, 80), jnp.int32),
            pltpu.VMEM((CB, EMBED), jnp.float32),
            pltpu.VMEM((CB * POS, EMBED), jnp.float32),
            pltpu.VMEM((CB * NEG, EMBED), jnp.float32),
            pltpu.VMEM((CB, EMBED), jnp.float32),
            pltpu.SemaphoreType.DMA,
        ],
    )(_sc_dots_kernel)
    dots = sc_call(cen_tensor, pos_flat, neg_flat, in_table, out_table)

    return pl.pallas_call(
        _tc_loss_kernel,
        out_shape=jax.ShapeDtypeStruct((BATCH,), jnp.float32),
    )(dots)
